# baseline (speedup 1.0000x reference)
import functools

import jax
import jax.numpy as jnp
from jax import lax
from jax.experimental import pallas as pl
from jax.experimental.pallas import tpu as pltpu
from jax.experimental.pallas import tpu_sc as plsc

_VOCAB = 100000
_MAXLEN = 200
_EMBED = 64
_EPAD = 128
_PITCH = 65
_BATCH = 4096
_N = _BATCH * _MAXLEN
_NC, _NS = 2, 16
_NW = _NC * _NS
_TPW = _N // _NW
_G = 128
_C = 256
_GPC = _C // _G
_NCH = _TPW // _C


def _sc_body(xf, psf, tok_hbm, ps_hbm, out_hbm,
             ix0, ix1, ip0, ip1, buf, ps_local, ps_exp,
             isem0, isem1, gsem0, gsem1, osem0, osem1):
    wid = lax.axis_index("s") * _NC + lax.axis_index("c")
    idx_x = (ix0, ix1)
    idx_p = (ip0, ip1)
    isem = (isem0, isem1)
    gsem = (gsem0, gsem1)
    osem = (osem0, osem1)

    pltpu.sync_copy(ps_hbm, ps_local)

    def do_idx(c, p):
        base = wid * _TPW + c * _C
        pltpu.async_copy(xf.at[pl.ds(base, _C)], idx_x[p], isem[p])
        pltpu.async_copy(psf.at[pl.ds(base, _C)], idx_p[p], isem[p])

    def wait_idx(p):
        pltpu.make_async_copy(xf.at[pl.ds(0, _C)], idx_x[p], isem[p]).wait()
        pltpu.make_async_copy(psf.at[pl.ds(0, _C)], idx_p[p], isem[p]).wait()

    def do_gather(p):
        for g in range(_GPC):
            rows = pl.ds(g * _G, _G)
            pltpu.async_copy(tok_hbm.at[idx_x[p].at[rows]],
                             buf.at[p, rows], gsem[p])

    def wait_gather(p):
        for g in range(_GPC):
            rows = pl.ds(g * _G, _G)
            pltpu.make_async_copy(tok_hbm.at[pl.ds(0, _G)],
                                  buf.at[p, rows], gsem[p]).wait()

    def do_scatter(c, p):
        base = wid * _TPW + c * _C
        pltpu.async_copy(buf.at[p], out_hbm.at[pl.ds(base, _C)], osem[p])

    def wait_scatter(p):
        pltpu.make_async_copy(buf.at[p], out_hbm.at[pl.ds(0, _C)],
                              osem[p]).wait()

    do_idx(0, 0)
    wait_idx(0)
    do_gather(0)
    do_idx(1, 1)

    lanes = lax.iota(jnp.int32, 16)

    def half(c, p):
        q = 1 - p

        @pl.when(c + 1 < _NCH)
        def _():
            wait_idx(q)

            @pl.when(c >= 1)
            def _():
                wait_scatter(q)

            do_gather(q)

        def expand(g16, carry2):
            t0 = g16 * 16
            r65 = idx_p[p][pl.ds(t0, 16)] * _PITCH
            t65 = (t0 + lanes) * _PITCH
            for col in range(_EMBED):
                vals = plsc.load_gather(ps_local, [r65 + col])
                plsc.store_scatter(ps_exp, [t65 + col], vals)
            return carry2

        lax.fori_loop(0, _C // 16, expand, 0)

        @pl.when(c + 2 < _NCH)
        def _():
            do_idx(c + 2, p)

        wait_gather(p)

        def add_tok(t, carry2):
            t65 = t * _PITCH
            for j in range(_EMBED // 16):
                plsc.addupdate(buf.at[p, t, pl.ds(j * 16, 16)],
                               ps_exp[pl.ds(t65 + j * 16, 16)])
            return carry2

        lax.fori_loop(0, _C, add_tok, 0, unroll=4)
        do_scatter(c, p)

    def pair(c2, carry):
        half(2 * c2, 0)
        half(2 * c2 + 1, 1)
        return carry

    lax.fori_loop(0, _NCH // 2, pair, 0)
    wait_scatter(0)
    wait_scatter(1)


@functools.partial(jax.jit, static_argnames=())
def _launch(xf, psf, tok128, ps65):
    mesh = plsc.VectorSubcoreMesh(core_axis_name="c", subcore_axis_name="s")
    return pl.kernel(
        _sc_body,
        out_type=jax.ShapeDtypeStruct((_N, _EPAD), jnp.float32),
        mesh=mesh,
        scratch_types=[
            pltpu.VMEM((_C,), jnp.int32),
            pltpu.VMEM((_C,), jnp.int32),
            pltpu.VMEM((_C,), jnp.int32),
            pltpu.VMEM((_C,), jnp.int32),
            pltpu.VMEM((2, _C, _EPAD), jnp.float32),
            pltpu.VMEM((2 * _MAXLEN * _PITCH,), jnp.float32),
            pltpu.VMEM((_C * _PITCH,), jnp.float32),
            pltpu.SemaphoreType.DMA,
            pltpu.SemaphoreType.DMA,
            pltpu.SemaphoreType.DMA,
            pltpu.SemaphoreType.DMA,
            pltpu.SemaphoreType.DMA,
            pltpu.SemaphoreType.DMA,
        ],
        compiler_params=pltpu.CompilerParams(use_tc_tiling_on_sc=True,
                                             needs_layout_passes=False),
    )(xf, psf, tok128, ps65)


def kernel(x, segment_ids, token_table, pos_table, seg_table):
    xf = x.astype(jnp.int32).reshape(_N)
    positions = jnp.arange(_MAXLEN, dtype=jnp.int32)
    psf = (segment_ids.astype(jnp.int32) * _MAXLEN
           + positions[None, :]).reshape(_N)
    tok128 = jnp.pad(token_table, ((0, 0), (0, _EPAD - _EMBED)))
    ps65 = jnp.pad(
        (seg_table[:, None, :] + pos_table[None, :, :]).reshape(
            2 * _MAXLEN, _EMBED), ((0, 0), (0, _PITCH - _EMBED))).reshape(-1)
    out = _launch(xf, psf, tok128, ps65)
    return out[:, :_EMBED].reshape(_BATCH, _MAXLEN, _EMBED)


# speedup vs baseline: 1.2357x; 1.2357x over previous
import functools

import jax
import jax.numpy as jnp
from jax import lax
from jax.experimental import pallas as pl
from jax.experimental.pallas import tpu as pltpu
from jax.experimental.pallas import tpu_sc as plsc

_VOCAB = 100000
_MAXLEN = 200
_EMBED = 64
_EPAD = 128
_BATCH = 4096
_N = _BATCH * _MAXLEN
_NC, _NS = 2, 16
_NW = _NC * _NS
_TPW = _N // _NW
_G = 128
_C = 128
_GPC = _C // _G
_NCH = _TPW // _C


def _sc_body(xf, psf, tok_hbm, ps_hbm, out_hbm,
             ix0, ix1, ip0, ip1, buf, buf_b,
             isem0, isem1, gsem0, gsem1, osem0, osem1):
    wid = lax.axis_index("s") * _NC + lax.axis_index("c")
    idx_x = (ix0, ix1)
    idx_p = (ip0, ip1)
    isem = (isem0, isem1)
    gsem = (gsem0, gsem1)
    osem = (osem0, osem1)

    def do_idx(c, p):
        base = wid * _TPW + c * _C
        pltpu.async_copy(xf.at[pl.ds(base, _C)], idx_x[p], isem[p])
        pltpu.async_copy(psf.at[pl.ds(base, _C)], idx_p[p], isem[p])

    def wait_idx(p):
        pltpu.make_async_copy(xf.at[pl.ds(0, _C)], idx_x[p], isem[p]).wait()
        pltpu.make_async_copy(psf.at[pl.ds(0, _C)], idx_p[p], isem[p]).wait()

    def do_gather(p):
        for g in range(_GPC):
            rows = pl.ds(g * _G, _G)
            pltpu.async_copy(tok_hbm.at[idx_x[p].at[rows]],
                             buf.at[p, rows], gsem[p])
            pltpu.async_copy(ps_hbm.at[idx_p[p].at[rows]],
                             buf_b.at[p, rows], gsem[p])

    def wait_gather(p):
        for g in range(_GPC):
            rows = pl.ds(g * _G, _G)
            pltpu.make_async_copy(tok_hbm.at[pl.ds(0, _G)],
                                  buf.at[p, rows], gsem[p]).wait()
            pltpu.make_async_copy(ps_hbm.at[pl.ds(0, _G)],
                                  buf_b.at[p, rows], gsem[p]).wait()

    def do_scatter(c, p):
        base = wid * _TPW + c * _C
        pltpu.async_copy(buf.at[p], out_hbm.at[pl.ds(base, _C)], osem[p])

    def wait_scatter(p):
        pltpu.make_async_copy(buf.at[p], out_hbm.at[pl.ds(0, _C)],
                              osem[p]).wait()

    do_idx(0, 0)
    wait_idx(0)
    do_gather(0)
    do_idx(1, 1)

    def half(c, p):
        q = 1 - p

        @pl.when(c + 1 < _NCH)
        def _():
            wait_idx(q)

            @pl.when(c >= 1)
            def _():
                wait_scatter(q)

            do_gather(q)

        wait_gather(p)

        @pl.when(c + 2 < _NCH)
        def _():
            do_idx(c + 2, p)

        def add_tok(t, carry2):
            for j in range(_EMBED // 16):
                col = pl.ds(j * 16, 16)
                plsc.addupdate(buf.at[p, t, col], buf_b[p, t, col])
            return carry2

        lax.fori_loop(0, _C, add_tok, 0, unroll=4)
        do_scatter(c, p)

    def pair(c2, carry):
        half(2 * c2, 0)
        half(2 * c2 + 1, 1)
        return carry

    lax.fori_loop(0, _NCH // 2, pair, 0)
    wait_scatter(0)
    wait_scatter(1)


@functools.partial(jax.jit, static_argnames=())
def _launch(xf, psf, tok128, ps128):
    mesh = plsc.VectorSubcoreMesh(core_axis_name="c", subcore_axis_name="s")
    return pl.kernel(
        _sc_body,
        out_type=jax.ShapeDtypeStruct((_N, _EPAD), jnp.float32),
        mesh=mesh,
        scratch_types=[
            pltpu.VMEM((_C,), jnp.int32),
            pltpu.VMEM((_C,), jnp.int32),
            pltpu.VMEM((_C,), jnp.int32),
            pltpu.VMEM((_C,), jnp.int32),
            pltpu.VMEM((2, _C, _EPAD), jnp.float32),
            pltpu.VMEM((2, _C, _EPAD), jnp.float32),
            pltpu.SemaphoreType.DMA,
            pltpu.SemaphoreType.DMA,
            pltpu.SemaphoreType.DMA,
            pltpu.SemaphoreType.DMA,
            pltpu.SemaphoreType.DMA,
            pltpu.SemaphoreType.DMA,
        ],
        compiler_params=pltpu.CompilerParams(use_tc_tiling_on_sc=True),
    )(xf, psf, tok128, ps128)


def kernel(x, segment_ids, token_table, pos_table, seg_table):
    xf = x.astype(jnp.int32).reshape(_N)
    positions = jnp.arange(_MAXLEN, dtype=jnp.int32)
    psf = (segment_ids.astype(jnp.int32) * _MAXLEN
           + positions[None, :]).reshape(_N)
    tok128 = jnp.pad(token_table, ((0, 0), (0, _EPAD - _EMBED)))
    ps128 = jnp.pad(
        (seg_table[:, None, :] + pos_table[None, :, :]).reshape(
            2 * _MAXLEN, _EMBED), ((0, 0), (0, _EPAD - _EMBED)))
    out = _launch(xf, psf, tok128, ps128)
    return out[:, :_EMBED].reshape(_BATCH, _MAXLEN, _EMBED)


# speedup vs baseline: 1.2400x; 1.0034x over previous
import functools

import jax
import jax.numpy as jnp
from jax import lax
from jax.experimental import pallas as pl
from jax.experimental.pallas import tpu as pltpu
from jax.experimental.pallas import tpu_sc as plsc

_VOCAB = 100000
_MAXLEN = 200
_EMBED = 64
_EPAD = 128
_BATCH = 4096
_N = _BATCH * _MAXLEN
_NC, _NS = 2, 16
_NW = _NC * _NS
_TPW = _N // _NW
_G = 128
_C = 128
_GPC = _C // _G
_NCH = _TPW // _C


def _sc_body(xf, psf, tok_hbm, ps_hbm, out_hbm,
             ix0, ix1, ip0, ip1, buf, buf_b,
             isem0, isem1, gsem0, gsem1, osem0, osem1):
    wid = lax.axis_index("s") * _NC + lax.axis_index("c")
    idx_x = (ix0, ix1)
    idx_p = (ip0, ip1)
    isem = (isem0, isem1)
    gsem = (gsem0, gsem1)
    osem = (osem0, osem1)

    def do_idx(c, p):
        base = wid * _TPW + c * _C
        pltpu.async_copy(xf.at[pl.ds(base, _C)], idx_x[p], isem[p])
        pltpu.async_copy(psf.at[pl.ds(base, _C)], idx_p[p], isem[p])

    def wait_idx(p):
        pltpu.make_async_copy(xf.at[pl.ds(0, _C)], idx_x[p], isem[p]).wait()
        pltpu.make_async_copy(psf.at[pl.ds(0, _C)], idx_p[p], isem[p]).wait()

    def do_gather(p):
        for g in range(_GPC):
            rows = pl.ds(g * _G, _G)
            pltpu.async_copy(tok_hbm.at[idx_x[p].at[rows]],
                             buf.at[p, rows], gsem[p])
            pltpu.async_copy(ps_hbm.at[idx_p[p].at[rows]],
                             buf_b.at[p, rows], gsem[p])

    def wait_gather(p):
        for g in range(_GPC):
            rows = pl.ds(g * _G, _G)
            pltpu.make_async_copy(tok_hbm.at[pl.ds(0, _G)],
                                  buf.at[p, rows], gsem[p]).wait()
            pltpu.make_async_copy(ps_hbm.at[pl.ds(0, _G)],
                                  buf_b.at[p, rows], gsem[p]).wait()

    def do_scatter(c, p):
        base = wid * _TPW + c * _C
        pltpu.async_copy(buf.at[p], out_hbm.at[pl.ds(base, _C)], osem[p])

    def wait_scatter(p):
        pltpu.make_async_copy(buf.at[p], out_hbm.at[pl.ds(0, _C)],
                              osem[p]).wait()

    do_idx(0, 0)
    wait_idx(0)
    do_gather(0)
    do_idx(1, 1)

    def half(c, p):
        q = 1 - p

        @pl.when(c + 1 < _NCH)
        def _():
            wait_idx(q)

            @pl.when(c >= 1)
            def _():
                wait_scatter(q)

            do_gather(q)

        wait_gather(p)

        @pl.when(c + 2 < _NCH)
        def _():
            do_idx(c + 2, p)

        def add_tok(t, carry2):
            for j in range(_EMBED // 16):
                col = pl.ds(j * 16, 16)
                plsc.addupdate(buf.at[p, t, col], buf_b[p, t, col])
            return carry2

        lax.fori_loop(0, _C, add_tok, 0, unroll=4)
        do_scatter(c, p)

    def pair(c2, carry):
        half(2 * c2, 0)
        half(2 * c2 + 1, 1)
        return carry

    lax.fori_loop(0, _NCH // 2, pair, 0)
    wait_scatter(0)
    wait_scatter(1)


@functools.partial(jax.jit, static_argnames=())
def _launch(xf, psf, tok128, ps128):
    mesh = plsc.VectorSubcoreMesh(core_axis_name="c", subcore_axis_name="s")
    return pl.kernel(
        _sc_body,
        out_type=jax.ShapeDtypeStruct((_N, _EPAD), jnp.float32),
        mesh=mesh,
        scratch_types=[
            pltpu.VMEM((_C,), jnp.int32),
            pltpu.VMEM((_C,), jnp.int32),
            pltpu.VMEM((_C,), jnp.int32),
            pltpu.VMEM((_C,), jnp.int32),
            pltpu.VMEM((2, _C, _EPAD), jnp.float32),
            pltpu.VMEM((2, _C, _EPAD), jnp.float32),
            pltpu.SemaphoreType.DMA,
            pltpu.SemaphoreType.DMA,
            pltpu.SemaphoreType.DMA,
            pltpu.SemaphoreType.DMA,
            pltpu.SemaphoreType.DMA,
            pltpu.SemaphoreType.DMA,
        ],
        compiler_params=pltpu.CompilerParams(use_tc_tiling_on_sc=True),
    )(xf, psf, tok128, ps128)


def kernel(x, segment_ids, token_table, pos_table, seg_table):
    xf = x.astype(jnp.int32).reshape(_N)
    positions = jnp.arange(_MAXLEN, dtype=jnp.int32)
    psf = (segment_ids.astype(jnp.int32) * _MAXLEN
           + positions[None, :]).reshape(_N)
    tok128 = jnp.pad(token_table, ((0, 0), (0, _EPAD - _EMBED)))
    ps128 = jnp.pad(
        (seg_table[:, None, :] + pos_table[None, :, :]).reshape(
            2 * _MAXLEN, _EMBED), ((0, 0), (0, _EPAD - _EMBED)))
    out = _launch(xf, psf, tok128, ps128)
    return out.reshape(_BATCH, _MAXLEN, _EPAD)[:, :, :_EMBED]
